# baseline (speedup 1.0000x reference)
import functools

import jax
import jax.numpy as jnp
from jax import lax
from jax.experimental import pallas as pl
from jax.experimental.pallas import tpu as pltpu
from jax.experimental.pallas import tpu_sc as plsc

N = 50000
P = 512
D = 64
C = 128
W = 500.0


def _sc_gather(combined, peak_indices):
    info = plsc.get_sparse_core_info()
    nc, ns = info.num_cores, info.num_subcores
    nw = nc * ns
    bpw = P // nw

    mesh = plsc.VectorSubcoreMesh(core_axis_name="c", subcore_axis_name="s")

    @functools.partial(
        pl.kernel,
        mesh=mesh,
        out_type=jax.ShapeDtypeStruct((P, C), jnp.float32),
        scratch_types=[
            pltpu.VMEM((bpw,), jnp.int32),
            pltpu.VMEM((bpw, C), jnp.float32),
            pltpu.SemaphoreType.DMA,
        ],
    )
    def k(table_hbm, idx_hbm, out_hbm, idx_v, rows_v, sem):
        wid = lax.axis_index("s") * nc + lax.axis_index("c")
        base = wid * bpw
        pltpu.sync_copy(idx_hbm.at[pl.ds(base, bpw)], idx_v)
        pltpu.async_copy(table_hbm.at[idx_v], rows_v, sem).wait()
        pltpu.sync_copy(rows_v, out_hbm.at[pl.ds(base, bpw)])

    return k(combined, peak_indices)


def _tc_body(comb_ref, g_ref, confT_ref, out_ref, cfT_ref, rhs_ref):
    i = pl.program_id(0)

    @pl.when(i == 0)
    def _():
        g = g_ref[...]
        pr = g[:, 0:D]
        ps = jnp.sum(pr * pr, axis=1, keepdims=True)
        prn = pr * lax.rsqrt(jnp.maximum(ps, 1e-24))
        cfT_ref[...] = prn.T * confT_ref[...]
        pT = g[:, D:D + 8].T
        pm = pT[0:4, :]
        b2 = jnp.sum(pm * pm, axis=0, keepdims=True)
        rhs_ref[...] = jnp.concatenate(
            [-2.0 * pm, b2, jnp.ones_like(b2)], axis=0)

    v = comb_ref[...]
    x = v[:, 0:D]
    s = jnp.sum(x * x, axis=1, keepdims=True)
    xn = x * lax.rsqrt(jnp.maximum(s, 1e-24))
    logits = jnp.dot(xn, cfT_ref[...],
                     preferred_element_type=jnp.float32)

    c5 = v[:, D:D + 5]
    c4 = v[:, D:D + 4]
    a2 = jnp.sum(c4 * c4, axis=1, keepdims=True)
    lhs = jnp.concatenate([c5, a2], axis=1)
    d2 = jnp.dot(lhs, rhs_ref[...],
                 preferred_element_type=jnp.float32)
    dist = jnp.maximum(jnp.sqrt(d2), 0.1)
    dmin = jnp.min(dist, axis=1, keepdims=True)
    e = jnp.exp(dmin - dist)
    r = 1.0 / jnp.maximum(jnp.sum(e, axis=1, keepdims=True), 1e-30)
    outv = jnp.clip(logits * (e * r), -10.0, 10.0)
    same = d2 < (W * W * 0.5)
    out_ref[...] = jnp.where(same, outv, -jnp.inf)


def _tc_affinity(combined, g, confT, block_n):
    grid = (N // block_n,)
    return pl.pallas_call(
        _tc_body,
        grid=grid,
        in_specs=[
            pl.BlockSpec((block_n, C), lambda i: (i, 0)),
            pl.BlockSpec((P, C), lambda i: (0, 0)),
            pl.BlockSpec((1, P), lambda i: (0, 0)),
        ],
        out_specs=pl.BlockSpec((block_n, P), lambda i: (i, 0)),
        out_shape=jax.ShapeDtypeStruct((N, P), jnp.float32),
        scratch_shapes=[pltpu.VMEM((D, P), jnp.float32),
                        pltpu.VMEM((6, P), jnp.float32)],
        compiler_params=pltpu.CompilerParams(
            dimension_semantics=("arbitrary",),
        ),
    )(combined, g, confT)


def kernel(voxel_feats, centroid_confidences, batch_ids, spatial_coords,
           peak_indices):
    combined = jnp.concatenate(
        [voxel_feats,
         spatial_coords.astype(jnp.float32),
         batch_ids[:, None].astype(jnp.float32) * W,
         jnp.ones((N, 1), jnp.float32),
         jnp.zeros((N, C - D - 5), jnp.float32)], axis=1)
    g = _sc_gather(combined, peak_indices)
    confT = centroid_confidences.T
    return _tc_affinity(combined, g, confT, block_n=1000)


# speedup vs baseline: 1.0027x; 1.0027x over previous
import functools

import jax
import jax.numpy as jnp
from jax import lax
from jax.experimental import pallas as pl
from jax.experimental.pallas import tpu as pltpu
from jax.experimental.pallas import tpu_sc as plsc

N = 50000
P = 512
D = 64
C = 128
W = 500.0


def _sc_gather(combined, peak_indices):
    info = plsc.get_sparse_core_info()
    nc, ns = info.num_cores, info.num_subcores
    nw = nc * ns
    bpw = P // nw

    mesh = plsc.VectorSubcoreMesh(core_axis_name="c", subcore_axis_name="s")

    @functools.partial(
        pl.kernel,
        mesh=mesh,
        out_type=jax.ShapeDtypeStruct((P, C), jnp.float32),
        scratch_types=[
            pltpu.VMEM((bpw,), jnp.int32),
            pltpu.VMEM((bpw, C), jnp.float32),
            pltpu.SemaphoreType.DMA,
        ],
        compiler_params=pltpu.CompilerParams(use_tc_tiling_on_sc=True),
    )
    def k(table_hbm, idx_hbm, out_hbm, idx_v, rows_v, sem):
        wid = lax.axis_index("s") * nc + lax.axis_index("c")
        base = wid * bpw
        pltpu.sync_copy(idx_hbm.at[pl.ds(base, bpw)], idx_v)
        pltpu.async_copy(table_hbm.at[idx_v], rows_v, sem).wait()
        pltpu.sync_copy(rows_v, out_hbm.at[pl.ds(base, bpw)])

    return k(combined, peak_indices)


def _tc_body(comb_ref, g_ref, confT_ref, out_ref, cfT_ref, rhs_ref):
    i = pl.program_id(0)

    @pl.when(i == 0)
    def _():
        g = g_ref[...]
        pr = g[:, 0:D]
        ps = jnp.sum(pr * pr, axis=1, keepdims=True)
        prn = pr * lax.rsqrt(jnp.maximum(ps, 1e-24))
        cfT_ref[...] = prn.T * confT_ref[...]
        pT = g[:, D:D + 8].T
        pm = pT[0:4, :]
        b2 = jnp.sum(pm * pm, axis=0, keepdims=True)
        rhs_ref[...] = jnp.concatenate(
            [-2.0 * pm, b2, jnp.ones_like(b2)], axis=0)

    v = comb_ref[...]
    x = v[:, 0:D]
    s = jnp.sum(x * x, axis=1, keepdims=True)
    xn = x * lax.rsqrt(jnp.maximum(s, 1e-24))
    logits = jnp.dot(xn, cfT_ref[...],
                     preferred_element_type=jnp.float32)

    c5 = v[:, D:D + 5]
    c4 = v[:, D:D + 4]
    a2 = jnp.sum(c4 * c4, axis=1, keepdims=True)
    lhs = jnp.concatenate([c5, a2], axis=1)
    d2 = jnp.dot(lhs, rhs_ref[...],
                 preferred_element_type=jnp.float32)
    dist = jnp.maximum(jnp.sqrt(d2), 0.1)
    dmin = jnp.min(dist, axis=1, keepdims=True)
    e = jnp.exp(dmin - dist)
    r = 1.0 / jnp.maximum(jnp.sum(e, axis=1, keepdims=True), 1e-30)
    outv = jnp.clip(logits * (e * r), -10.0, 10.0)
    same = d2 < (W * W * 0.5)
    out_ref[...] = jnp.where(same, outv, -jnp.inf)


def _tc_affinity(combined, g, confT, block_n):
    grid = (N // block_n,)
    return pl.pallas_call(
        _tc_body,
        grid=grid,
        in_specs=[
            pl.BlockSpec((block_n, C), lambda i: (i, 0)),
            pl.BlockSpec((P, C), lambda i: (0, 0)),
            pl.BlockSpec((1, P), lambda i: (0, 0)),
        ],
        out_specs=pl.BlockSpec((block_n, P), lambda i: (i, 0)),
        out_shape=jax.ShapeDtypeStruct((N, P), jnp.float32),
        scratch_shapes=[pltpu.VMEM((D, P), jnp.float32),
                        pltpu.VMEM((6, P), jnp.float32)],
        compiler_params=pltpu.CompilerParams(
            dimension_semantics=("arbitrary",),
        ),
    )(combined, g, confT)


def kernel(voxel_feats, centroid_confidences, batch_ids, spatial_coords,
           peak_indices):
    combined = jnp.concatenate(
        [voxel_feats,
         spatial_coords.astype(jnp.float32),
         batch_ids[:, None].astype(jnp.float32) * W,
         jnp.ones((N, 1), jnp.float32),
         jnp.zeros((N, C - D - 5), jnp.float32)], axis=1)
    g = _sc_gather(combined, peak_indices)
    confT = centroid_confidences.T
    return _tc_affinity(combined, g, confT, block_n=1000)


# speedup vs baseline: 1.3998x; 1.3961x over previous
import functools

import jax
import jax.numpy as jnp
from jax import lax
from jax.experimental import pallas as pl
from jax.experimental.pallas import tpu as pltpu
from jax.experimental.pallas import tpu_sc as plsc

N = 50000
P = 512
D = 64
C = 128
W = 500.0


def _build_body(vf_ref, sc_ref, b_ref, out_ref, meta_ref):
    vf = vf_ref[...]
    cf = sc_ref[...].astype(jnp.float32)
    wb = b_ref[...].astype(jnp.float32) * W
    one = jnp.ones_like(wb)
    zer = jnp.zeros((vf.shape[0], 11), jnp.float32)
    meta = jnp.concatenate([cf, wb, one, zer], axis=1)
    meta_ref[...] = meta
    zer2 = jnp.zeros((vf.shape[0], C - D - 16), jnp.float32)
    out_ref[...] = jnp.concatenate([vf, meta, zer2], axis=1)


def _build_combined(voxel_feats, spatial_coords, bid2, block_n=2000):
    grid = (N // block_n,)
    return pl.pallas_call(
        _build_body,
        grid=grid,
        in_specs=[
            pl.BlockSpec((block_n, D), lambda i: (i, 0)),
            pl.BlockSpec((block_n, 3), lambda i: (i, 0)),
            pl.BlockSpec((block_n, 1), lambda i: (i, 0)),
        ],
        out_specs=[pl.BlockSpec((block_n, C), lambda i: (i, 0)),
                   pl.BlockSpec((block_n, 16), lambda i: (i, 0))],
        out_shape=[jax.ShapeDtypeStruct((N, C), jnp.float32),
                   jax.ShapeDtypeStruct((N, 16), jnp.float32)],
        compiler_params=pltpu.CompilerParams(
            dimension_semantics=("arbitrary",),
        ),
    )(voxel_feats, spatial_coords, bid2)


def _sc_gather(combined, peak_indices):
    info = plsc.get_sparse_core_info()
    nc, ns = info.num_cores, info.num_subcores
    nw = nc * ns
    bpw = P // nw

    mesh = plsc.VectorSubcoreMesh(core_axis_name="c", subcore_axis_name="s")

    @functools.partial(
        pl.kernel,
        mesh=mesh,
        out_type=jax.ShapeDtypeStruct((P, C), jnp.float32),
        scratch_types=[
            pltpu.VMEM((bpw,), jnp.int32),
            pltpu.VMEM((bpw, C), jnp.float32),
            pltpu.SemaphoreType.DMA,
        ],
        compiler_params=pltpu.CompilerParams(use_tc_tiling_on_sc=True),
    )
    def k(table_hbm, idx_hbm, out_hbm, idx_v, rows_v, sem):
        wid = lax.axis_index("s") * nc + lax.axis_index("c")
        base = wid * bpw
        pltpu.sync_copy(idx_hbm.at[pl.ds(base, bpw)], idx_v)
        pltpu.async_copy(table_hbm.at[idx_v], rows_v, sem).wait()
        pltpu.sync_copy(rows_v, out_hbm.at[pl.ds(base, bpw)])

    return k(combined, peak_indices)


def _tc_body(vf_ref, meta_ref, g_ref, confT_ref, out_ref, cfT_ref, rhs_ref):
    i = pl.program_id(0)

    @pl.when(i == 0)
    def _():
        g = g_ref[...]
        pr = g[:, 0:D]
        ps = jnp.sum(pr * pr, axis=1, keepdims=True)
        prn = pr * lax.rsqrt(jnp.maximum(ps, 1e-24))
        cfT_ref[...] = prn.T * confT_ref[...]
        pT = g[:, D:D + 8].T
        pm = pT[0:4, :]
        b2 = jnp.sum(pm * pm, axis=0, keepdims=True)
        rhs_ref[...] = jnp.concatenate(
            [-2.0 * pm, b2, jnp.ones_like(b2)], axis=0)

    x = vf_ref[...]
    s = jnp.sum(x * x, axis=1, keepdims=True)
    xn = x * lax.rsqrt(jnp.maximum(s, 1e-24))
    logits = jnp.dot(xn, cfT_ref[...],
                     preferred_element_type=jnp.float32)

    mf = meta_ref[...]
    c5 = mf[:, 0:5]
    c4 = mf[:, 0:4]
    a2 = jnp.sum(c4 * c4, axis=1, keepdims=True)
    lhs = jnp.concatenate([c5, a2], axis=1)
    d2 = jnp.dot(lhs, rhs_ref[...],
                 preferred_element_type=jnp.float32)
    dist = jnp.maximum(jnp.sqrt(d2), 0.1)
    dmin = jnp.min(dist, axis=1, keepdims=True)
    e = jnp.exp(dmin - dist)
    r = 1.0 / jnp.maximum(jnp.sum(e, axis=1, keepdims=True), 1e-30)
    outv = jnp.clip(logits * (e * r), -10.0, 10.0)
    same = d2 < (W * W * 0.5)
    out_ref[...] = jnp.where(same, outv, -jnp.inf)


def _tc_affinity(voxel_feats, meta16, g, confT, block_n):
    grid = (N // block_n,)
    return pl.pallas_call(
        _tc_body,
        grid=grid,
        in_specs=[
            pl.BlockSpec((block_n, D), lambda i: (i, 0)),
            pl.BlockSpec((block_n, 16), lambda i: (i, 0)),
            pl.BlockSpec((P, C), lambda i: (0, 0)),
            pl.BlockSpec((1, P), lambda i: (0, 0)),
        ],
        out_specs=pl.BlockSpec((block_n, P), lambda i: (i, 0)),
        out_shape=jax.ShapeDtypeStruct((N, P), jnp.float32),
        scratch_shapes=[pltpu.VMEM((D, P), jnp.float32),
                        pltpu.VMEM((6, P), jnp.float32)],
        compiler_params=pltpu.CompilerParams(
            dimension_semantics=("arbitrary",),
        ),
    )(voxel_feats, meta16, g, confT)


def kernel(voxel_feats, centroid_confidences, batch_ids, spatial_coords,
           peak_indices):
    bid2 = batch_ids[:, None]
    combined, meta16 = _build_combined(voxel_feats, spatial_coords, bid2)
    g = _sc_gather(combined, peak_indices)
    confT = centroid_confidences.T
    return _tc_affinity(voxel_feats, meta16, g, confT, block_n=1000)


# speedup vs baseline: 2.0385x; 1.4563x over previous
import functools

import jax
import jax.numpy as jnp
from jax import lax
from jax.experimental import pallas as pl
from jax.experimental.pallas import tpu as pltpu
from jax.experimental.pallas import tpu_sc as plsc

N = 50000
P = 512
D = 64
C = 128
W = 500.0


def _build_body(vfT_ref, scT_ref, bT_ref, comb_ref, metaT_ref):
    vfT = vfT_ref[...]
    cf = scT_ref[...].astype(jnp.float32)
    wb = bT_ref[...].astype(jnp.float32) * W
    one = jnp.ones_like(wb)
    a2 = (jnp.sum(cf * cf, axis=0, keepdims=True) + wb * wb)
    zer = jnp.zeros((2, vfT.shape[1]), jnp.float32)
    metaT = jnp.concatenate([cf, wb, one, a2, zer], axis=0)
    metaT_ref[...] = metaT
    bb = vfT.shape[1]
    zer2 = jnp.zeros((bb, C - D - 8), jnp.float32)
    comb_ref[...] = jnp.concatenate(
        [vfT.T, metaT.T, zer2], axis=1)


def _build_combined(vfT, scT, bT, block_n=2048):
    grid = (pl.cdiv(N, block_n),)
    return pl.pallas_call(
        _build_body,
        grid=grid,
        in_specs=[
            pl.BlockSpec((D, block_n), lambda i: (0, i)),
            pl.BlockSpec((3, block_n), lambda i: (0, i)),
            pl.BlockSpec((1, block_n), lambda i: (0, i)),
        ],
        out_specs=[pl.BlockSpec((block_n, C), lambda i: (i, 0)),
                   pl.BlockSpec((8, block_n), lambda i: (0, i))],
        out_shape=[jax.ShapeDtypeStruct((N, C), jnp.float32),
                   jax.ShapeDtypeStruct((8, N), jnp.float32)],
        compiler_params=pltpu.CompilerParams(
            dimension_semantics=("arbitrary",),
        ),
    )(vfT, scT, bT)


def _sc_gather(combined, peak_indices):
    info = plsc.get_sparse_core_info()
    nc, ns = info.num_cores, info.num_subcores
    nw = nc * ns
    bpw = P // nw

    mesh = plsc.VectorSubcoreMesh(core_axis_name="c", subcore_axis_name="s")

    @functools.partial(
        pl.kernel,
        mesh=mesh,
        out_type=jax.ShapeDtypeStruct((P, C), jnp.float32),
        scratch_types=[
            pltpu.VMEM((bpw,), jnp.int32),
            pltpu.VMEM((bpw, C), jnp.float32),
            pltpu.SemaphoreType.DMA,
        ],
        compiler_params=pltpu.CompilerParams(use_tc_tiling_on_sc=True),
    )
    def k(table_hbm, idx_hbm, out_hbm, idx_v, rows_v, sem):
        wid = lax.axis_index("s") * nc + lax.axis_index("c")
        base = wid * bpw
        pltpu.sync_copy(idx_hbm.at[pl.ds(base, bpw)], idx_v)
        pltpu.async_copy(table_hbm.at[idx_v], rows_v, sem).wait()
        pltpu.sync_copy(rows_v, out_hbm.at[pl.ds(base, bpw)])

    return k(combined, peak_indices)


def _tc_body(vfT_ref, metaT_ref, g_ref, confT_ref, out_ref, cfT_ref, rhs_ref):
    i = pl.program_id(0)

    @pl.when(i == 0)
    def _():
        g = g_ref[...]
        pr = g[:, 0:D]
        ps = jnp.sum(pr * pr, axis=1, keepdims=True)
        prn = pr * lax.rsqrt(jnp.maximum(ps, 1e-24))
        cfT_ref[...] = prn.T * confT_ref[...]
        pT = g[:, D:D + 8].T
        pm = pT[0:4, :]
        b2 = jnp.sum(pm * pm, axis=0, keepdims=True)
        rhs_ref[...] = jnp.concatenate(
            [-2.0 * pm, b2, jnp.ones_like(b2)], axis=0)

    x = vfT_ref[...]
    s = jnp.sum(x * x, axis=0, keepdims=True)
    xn = x * lax.rsqrt(jnp.maximum(s, 1e-24))
    logits = lax.dot_general(
        xn, cfT_ref[...], (((0,), (0,)), ((), ())),
        preferred_element_type=jnp.float32)

    lhsT = metaT_ref[0:6, :]
    d2 = lax.dot_general(
        lhsT, rhs_ref[...], (((0,), (0,)), ((), ())),
        preferred_element_type=jnp.float32)
    dist = jnp.maximum(jnp.sqrt(d2), 0.1)
    dmin = jnp.min(dist, axis=1, keepdims=True)
    e = jnp.exp(dmin - dist)
    r = 1.0 / jnp.maximum(jnp.sum(e, axis=1, keepdims=True), 1e-30)
    outv = jnp.clip(logits * (e * r), -10.0, 10.0)
    same = d2 < (W * W * 0.5)
    out_ref[...] = jnp.where(same, outv, -jnp.inf)


def _tc_affinity(vfT, metaT, g, confT, block_n):
    grid = (pl.cdiv(N, block_n),)
    return pl.pallas_call(
        _tc_body,
        grid=grid,
        in_specs=[
            pl.BlockSpec((D, block_n), lambda i: (0, i)),
            pl.BlockSpec((8, block_n), lambda i: (0, i)),
            pl.BlockSpec((P, C), lambda i: (0, 0)),
            pl.BlockSpec((1, P), lambda i: (0, 0)),
        ],
        out_specs=pl.BlockSpec((block_n, P), lambda i: (i, 0)),
        out_shape=jax.ShapeDtypeStruct((N, P), jnp.float32),
        scratch_shapes=[pltpu.VMEM((D, P), jnp.float32),
                        pltpu.VMEM((6, P), jnp.float32)],
        compiler_params=pltpu.CompilerParams(
            dimension_semantics=("arbitrary",),
        ),
    )(vfT, metaT, g, confT)


def kernel(voxel_feats, centroid_confidences, batch_ids, spatial_coords,
           peak_indices):
    vfT = voxel_feats.T
    scT = spatial_coords.T
    bT = batch_ids[None, :]
    combined, metaT = _build_combined(vfT, scT, bT)
    g = _sc_gather(combined, peak_indices)
    confT = centroid_confidences.T
    return _tc_affinity(vfT, metaT, g, confT, block_n=1024)


# speedup vs baseline: 2.3450x; 1.1504x over previous
import functools

import jax
import jax.numpy as jnp
from jax import lax
from jax.experimental import pallas as pl
from jax.experimental.pallas import tpu as pltpu
from jax.experimental.pallas import tpu_sc as plsc

N = 50000
P = 512
D = 64
C = 128
W = 500.0


def _build_body(vfT_ref, scT_ref, bT_ref, comb_ref):
    vfT = vfT_ref[...]
    cf = scT_ref[...].astype(jnp.float32)
    wb = bT_ref[...].astype(jnp.float32) * W
    one = jnp.ones_like(wb)
    a2 = (jnp.sum(cf * cf, axis=0, keepdims=True) + wb * wb)
    metaT = jnp.concatenate([cf, wb, one, a2], axis=0)
    bb = vfT.shape[1]
    zer2 = jnp.zeros((bb, C - D - 6), jnp.float32)
    comb_ref[...] = jnp.concatenate(
        [vfT.T, metaT.T, zer2], axis=1)


def _build_combined(vfT, scT, bT, block_n=4096):
    grid = (pl.cdiv(N, block_n),)
    return pl.pallas_call(
        _build_body,
        grid=grid,
        in_specs=[
            pl.BlockSpec((D, block_n), lambda i: (0, i)),
            pl.BlockSpec((3, block_n), lambda i: (0, i)),
            pl.BlockSpec((1, block_n), lambda i: (0, i)),
        ],
        out_specs=pl.BlockSpec((block_n, C), lambda i: (i, 0)),
        out_shape=jax.ShapeDtypeStruct((N, C), jnp.float32),
        compiler_params=pltpu.CompilerParams(
            dimension_semantics=("arbitrary",),
        ),
    )(vfT, scT, bT)


def _sc_gather(combined, peak_indices):
    info = plsc.get_sparse_core_info()
    nc, ns = info.num_cores, info.num_subcores
    nw = nc * ns
    bpw = P // nw

    mesh = plsc.VectorSubcoreMesh(core_axis_name="c", subcore_axis_name="s")

    @functools.partial(
        pl.kernel,
        mesh=mesh,
        out_type=jax.ShapeDtypeStruct((P, C), jnp.float32),
        scratch_types=[
            pltpu.VMEM((bpw,), jnp.int32),
            pltpu.VMEM((bpw, C), jnp.float32),
            pltpu.SemaphoreType.DMA,
        ],
        compiler_params=pltpu.CompilerParams(use_tc_tiling_on_sc=True),
    )
    def k(table_hbm, idx_hbm, out_hbm, idx_v, rows_v, sem):
        wid = lax.axis_index("s") * nc + lax.axis_index("c")
        base = wid * bpw
        pltpu.sync_copy(idx_hbm.at[pl.ds(base, bpw)], idx_v)
        pltpu.async_copy(table_hbm.at[idx_v], rows_v, sem).wait()
        pltpu.sync_copy(rows_v, out_hbm.at[pl.ds(base, bpw)])

    return k(combined, peak_indices)


_LOG2E = 1.4426950408889634


def _tc_body(vfT_ref, scT_ref, bT_ref, g_ref, confT_ref, out_ref,
             cfT_ref, rhs_ref):
    i = pl.program_id(0)

    @pl.when(i == 0)
    def _():
        g = g_ref[...]
        pr = g[:, 0:D]
        ps = jnp.sum(pr * pr, axis=1, keepdims=True)
        prn = pr * lax.rsqrt(jnp.maximum(ps, 1e-24))
        cfT_ref[...] = prn.T * confT_ref[...]
        pT = g[:, D:D + 8].T
        pm = pT[0:4, :]
        b2 = pT[5:6, :]
        rhs_ref[...] = jnp.concatenate(
            [-2.0 * pm, b2, jnp.ones_like(b2)], axis=0)

    x = vfT_ref[...]
    s = jnp.sum(x * x, axis=0, keepdims=True)
    xn = x * lax.rsqrt(jnp.maximum(s, 1e-24))
    logits = lax.dot_general(
        xn, cfT_ref[...], (((0,), (0,)), ((), ())),
        preferred_element_type=jnp.float32)

    cf = scT_ref[...].astype(jnp.float32)
    wb = bT_ref[...].astype(jnp.float32) * W
    one = jnp.ones_like(wb)
    a2 = (jnp.sum(cf * cf, axis=0, keepdims=True) + wb * wb)
    lhsT = jnp.concatenate([cf, wb, one, a2], axis=0)
    d2 = lax.dot_general(
        lhsT, rhs_ref[...], (((0,), (0,)), ((), ())),
        preferred_element_type=jnp.float32)
    q = jnp.maximum(d2, 0.01)
    sdist = (q * _LOG2E) * lax.rsqrt(q)
    smin = jnp.min(sdist, axis=1, keepdims=True)
    e = jnp.exp2(smin - sdist)
    r = 1.0 / jnp.maximum(jnp.sum(e, axis=1, keepdims=True), 1e-30)
    outv = jnp.clip(logits * (e * r), -10.0, 10.0)
    same = d2 < (W * W * 0.5)
    out_ref[...] = jnp.where(same, outv, -jnp.inf)


def _tc_affinity(vfT, scT, bT, g, confT, block_n):
    grid = (pl.cdiv(N, block_n),)
    return pl.pallas_call(
        _tc_body,
        grid=grid,
        in_specs=[
            pl.BlockSpec((D, block_n), lambda i: (0, i)),
            pl.BlockSpec((3, block_n), lambda i: (0, i)),
            pl.BlockSpec((1, block_n), lambda i: (0, i)),
            pl.BlockSpec((P, C), lambda i: (0, 0)),
            pl.BlockSpec((1, P), lambda i: (0, 0)),
        ],
        out_specs=pl.BlockSpec((block_n, P), lambda i: (i, 0)),
        out_shape=jax.ShapeDtypeStruct((N, P), jnp.float32),
        scratch_shapes=[pltpu.VMEM((D, P), jnp.float32),
                        pltpu.VMEM((6, P), jnp.float32)],
        compiler_params=pltpu.CompilerParams(
            dimension_semantics=("arbitrary",),
        ),
    )(vfT, scT, bT, g, confT)


def kernel(voxel_feats, centroid_confidences, batch_ids, spatial_coords,
           peak_indices):
    vfT = voxel_feats.T
    scT = spatial_coords.T
    bT = batch_ids[None, :]
    combined = _build_combined(vfT, scT, bT)
    g = _sc_gather(combined, peak_indices)
    confT = centroid_confidences.T
    return _tc_affinity(vfT, scT, bT, g, confT, block_n=1024)


# speedup vs baseline: 2.4713x; 1.0538x over previous
import functools

import jax
import jax.numpy as jnp
from jax import lax
from jax.experimental import pallas as pl
from jax.experimental.pallas import tpu as pltpu
from jax.experimental.pallas import tpu_sc as plsc

N = 50000
P = 512
D = 64
C = 128
W = 500.0


def _build_body(vfT_ref, scT_ref, bT_ref, comb_ref):
    vfT = vfT_ref[...]
    cf = scT_ref[...].astype(jnp.float32)
    wb = bT_ref[...].astype(jnp.float32) * W
    one = jnp.ones_like(wb)
    a2 = (jnp.sum(cf * cf, axis=0, keepdims=True) + wb * wb)
    metaT = jnp.concatenate([cf, wb, one, a2], axis=0)
    bb = vfT.shape[1]
    zer2 = jnp.zeros((bb, C - D - 6), jnp.float32)
    comb_ref[...] = jnp.concatenate(
        [vfT.T, metaT.T, zer2], axis=1)


def _build_combined(vfT, scT, bT, block_n=4096):
    grid = (pl.cdiv(N, block_n),)
    return pl.pallas_call(
        _build_body,
        grid=grid,
        in_specs=[
            pl.BlockSpec((D, block_n), lambda i: (0, i)),
            pl.BlockSpec((3, block_n), lambda i: (0, i)),
            pl.BlockSpec((1, block_n), lambda i: (0, i)),
        ],
        out_specs=pl.BlockSpec((block_n, C), lambda i: (i, 0)),
        out_shape=jax.ShapeDtypeStruct((N, C), jnp.float32),
        compiler_params=pltpu.CompilerParams(
            dimension_semantics=("arbitrary",),
        ),
    )(vfT, scT, bT)


def _sc_gather(combined, peak_indices):
    info = plsc.get_sparse_core_info()
    nc, ns = info.num_cores, info.num_subcores
    nw = nc * ns
    bpw = P // nw

    mesh = plsc.VectorSubcoreMesh(core_axis_name="c", subcore_axis_name="s")

    @functools.partial(
        pl.kernel,
        mesh=mesh,
        out_type=jax.ShapeDtypeStruct((P, C), jnp.float32),
        scratch_types=[
            pltpu.VMEM((bpw,), jnp.int32),
            pltpu.VMEM((bpw, C), jnp.float32),
            pltpu.SemaphoreType.DMA,
        ],
        compiler_params=pltpu.CompilerParams(use_tc_tiling_on_sc=True),
    )
    def k(table_hbm, idx_hbm, out_hbm, idx_v, rows_v, sem):
        wid = lax.axis_index("s") * nc + lax.axis_index("c")
        base = wid * bpw
        pltpu.sync_copy(idx_hbm.at[pl.ds(base, bpw)], idx_v)
        pltpu.async_copy(table_hbm.at[idx_v], rows_v, sem).wait()
        pltpu.sync_copy(rows_v, out_hbm.at[pl.ds(base, bpw)])

    return k(combined, peak_indices)


_LOG2E = 1.4426950408889634


def _tc_body(vfT_ref, scT_ref, bT_ref, g_ref, confT_ref, out_ref,
             cfT_ref, rhs_ref):
    i = pl.program_id(0)

    @pl.when(i == 0)
    def _():
        g = g_ref[...]
        pr = g[:, 0:D]
        ps = jnp.sum(pr * pr, axis=1, keepdims=True)
        prn = pr * lax.rsqrt(jnp.maximum(ps, 1e-24))
        cfT_ref[...] = prn.T * confT_ref[...]
        pT = g[:, D:D + 8].T
        pm = pT[0:4, :]
        b2 = pT[5:6, :]
        rhs_ref[...] = jnp.concatenate(
            [-2.0 * pm, b2, jnp.ones_like(b2)], axis=0)

    x = vfT_ref[...]
    s = jnp.sum(x * x, axis=0, keepdims=True)
    xn = x * lax.rsqrt(jnp.maximum(s, 1e-24))
    logits = lax.dot_general(
        xn, cfT_ref[...], (((0,), (0,)), ((), ())),
        preferred_element_type=jnp.float32)

    cf = scT_ref[...].astype(jnp.float32)
    wb = bT_ref[...].astype(jnp.float32) * W
    one = jnp.ones_like(wb)
    a2 = (jnp.sum(cf * cf, axis=0, keepdims=True) + wb * wb)
    lhsT = jnp.concatenate([cf, wb, one, a2], axis=0)
    d2 = lax.dot_general(
        lhsT, rhs_ref[...], (((0,), (0,)), ((), ())),
        preferred_element_type=jnp.float32)
    q = jnp.maximum(d2, 0.01)
    sdist = (q * _LOG2E) * lax.rsqrt(q)
    smin = jnp.min(sdist, axis=1, keepdims=True)
    e = jnp.exp2(smin - sdist)
    r = 1.0 / jnp.maximum(jnp.sum(e, axis=1, keepdims=True), 1e-30)
    outv = jnp.clip(logits * (e * r), -10.0, 10.0)
    same = d2 < (W * W * 0.5)
    out_ref[...] = jnp.where(same, outv, -jnp.inf)


def _tc_affinity(vfT, scT, bT, g, confT, block_n):
    grid = (pl.cdiv(N, block_n),)
    return pl.pallas_call(
        _tc_body,
        grid=grid,
        in_specs=[
            pl.BlockSpec((D, block_n), lambda i: (0, i)),
            pl.BlockSpec((3, block_n), lambda i: (0, i)),
            pl.BlockSpec((1, block_n), lambda i: (0, i)),
            pl.BlockSpec((P, C), lambda i: (0, 0)),
            pl.BlockSpec((1, P), lambda i: (0, 0)),
        ],
        out_specs=pl.BlockSpec((block_n, P), lambda i: (i, 0)),
        out_shape=jax.ShapeDtypeStruct((N, P), jnp.float32),
        scratch_shapes=[pltpu.VMEM((D, P), jnp.float32),
                        pltpu.VMEM((6, P), jnp.float32)],
        compiler_params=pltpu.CompilerParams(
            dimension_semantics=("arbitrary",),
        ),
    )(vfT, scT, bT, g, confT)


def kernel(voxel_feats, centroid_confidences, batch_ids, spatial_coords,
           peak_indices):
    vfT = voxel_feats.T
    scT = spatial_coords.T
    bT = batch_ids[None, :]
    combined = _build_combined(vfT, scT, bT)
    g = _sc_gather(combined, peak_indices)
    confT = centroid_confidences.T
    return _tc_affinity(vfT, scT, bT, g, confT, block_n=2048)


# speedup vs baseline: 2.4809x; 1.0039x over previous
import functools

import jax
import jax.numpy as jnp
from jax import lax
from jax.experimental import pallas as pl
from jax.experimental.pallas import tpu as pltpu
from jax.experimental.pallas import tpu_sc as plsc

N = 50000
P = 512
D = 64
C = 128
W = 500.0


def _build_body(vfT_ref, scT_ref, bT_ref, comb_ref):
    vfT = vfT_ref[...]
    cf = scT_ref[...].astype(jnp.float32)
    wb = bT_ref[...].astype(jnp.float32) * W
    one = jnp.ones_like(wb)
    a2 = (jnp.sum(cf * cf, axis=0, keepdims=True) + wb * wb)
    metaT = jnp.concatenate([cf, wb, one, a2], axis=0)
    bb = vfT.shape[1]
    zer2 = jnp.zeros((bb, C - D - 6), jnp.float32)
    comb_ref[...] = jnp.concatenate(
        [vfT.T, metaT.T, zer2], axis=1)


def _build_combined(vfT, scT, bT, block_n=4096):
    grid = (pl.cdiv(N, block_n),)
    return pl.pallas_call(
        _build_body,
        grid=grid,
        in_specs=[
            pl.BlockSpec((D, block_n), lambda i: (0, i)),
            pl.BlockSpec((3, block_n), lambda i: (0, i)),
            pl.BlockSpec((1, block_n), lambda i: (0, i)),
        ],
        out_specs=pl.BlockSpec((block_n, C), lambda i: (i, 0)),
        out_shape=jax.ShapeDtypeStruct((N, C), jnp.float32),
        compiler_params=pltpu.CompilerParams(
            dimension_semantics=("arbitrary",),
        ),
    )(vfT, scT, bT)


def _sc_gather(combined, peak_indices):
    info = plsc.get_sparse_core_info()
    nc, ns = info.num_cores, info.num_subcores
    nw = nc * ns
    bpw = P // nw

    mesh = plsc.VectorSubcoreMesh(core_axis_name="c", subcore_axis_name="s")

    @functools.partial(
        pl.kernel,
        mesh=mesh,
        out_type=jax.ShapeDtypeStruct((P, C), jnp.float32),
        scratch_types=[
            pltpu.VMEM((bpw,), jnp.int32),
            pltpu.VMEM((bpw, C), jnp.float32),
            pltpu.SemaphoreType.DMA,
        ],
        compiler_params=pltpu.CompilerParams(use_tc_tiling_on_sc=True),
    )
    def k(table_hbm, idx_hbm, out_hbm, idx_v, rows_v, sem):
        wid = lax.axis_index("s") * nc + lax.axis_index("c")
        base = wid * bpw
        pltpu.sync_copy(idx_hbm.at[pl.ds(base, bpw)], idx_v)
        pltpu.async_copy(table_hbm.at[idx_v], rows_v, sem).wait()
        pltpu.sync_copy(rows_v, out_hbm.at[pl.ds(base, bpw)])

    return k(combined, peak_indices)


_LOG2E = 1.4426950408889634


def _tc_body(vfT_ref, scT_ref, bT_ref, g_ref, confT_ref, out_ref,
             cfT_ref, rhs_ref):
    i = pl.program_id(0)

    @pl.when(i == 0)
    def _():
        g = g_ref[...]
        pr = g[:, 0:D]
        ps = jnp.sum(pr * pr, axis=1, keepdims=True)
        prn = pr * lax.rsqrt(jnp.maximum(ps, 1e-24))
        cfT_ref[...] = prn.T * confT_ref[...]
        pT = g[:, D:D + 8].T
        pm = pT[0:4, :]
        b2 = pT[5:6, :]
        rhs_ref[...] = jnp.concatenate(
            [-2.0 * pm, b2, jnp.ones_like(b2)], axis=0)

    x = vfT_ref[...]
    s = jnp.sum(x * x, axis=0, keepdims=True)
    xn = x * lax.rsqrt(jnp.maximum(s, 1e-24))
    logits = lax.dot_general(
        xn, cfT_ref[...], (((0,), (0,)), ((), ())),
        preferred_element_type=jnp.float32)

    cf = scT_ref[...].astype(jnp.float32)
    wb = bT_ref[...].astype(jnp.float32) * W
    one = jnp.ones_like(wb)
    a2 = (jnp.sum(cf * cf, axis=0, keepdims=True) + wb * wb)
    lhsT = jnp.concatenate([cf, wb, one, a2], axis=0)
    d2 = lax.dot_general(
        lhsT, rhs_ref[...], (((0,), (0,)), ((), ())),
        preferred_element_type=jnp.float32)
    q = jnp.maximum(d2, 0.01)
    sdist = (q * _LOG2E) * lax.rsqrt(q)
    smin = jnp.min(sdist, axis=1, keepdims=True)
    e = jnp.exp2(smin - sdist)
    r = 1.0 / jnp.maximum(jnp.sum(e, axis=1, keepdims=True), 1e-30)
    outv = jnp.clip(logits * (e * r), -10.0, 10.0)
    same = d2 < (W * W * 0.5)
    out_ref[...] = jnp.where(same, outv, -jnp.inf)


def _tc_affinity(vfT, scT, bT, g, confT, block_n):
    grid = (pl.cdiv(N, block_n),)
    return pl.pallas_call(
        _tc_body,
        grid=grid,
        in_specs=[
            pl.BlockSpec((D, block_n), lambda i: (0, i)),
            pl.BlockSpec((3, block_n), lambda i: (0, i)),
            pl.BlockSpec((1, block_n), lambda i: (0, i)),
            pl.BlockSpec((P, C), lambda i: (0, 0)),
            pl.BlockSpec((1, P), lambda i: (0, 0)),
        ],
        out_specs=pl.BlockSpec((block_n, P), lambda i: (i, 0)),
        out_shape=jax.ShapeDtypeStruct((N, P), jnp.float32),
        scratch_shapes=[pltpu.VMEM((D, P), jnp.float32),
                        pltpu.VMEM((6, P), jnp.float32)],
        compiler_params=pltpu.CompilerParams(
            dimension_semantics=("arbitrary",),
        ),
    )(vfT, scT, bT, g, confT)


def kernel(voxel_feats, centroid_confidences, batch_ids, spatial_coords,
           peak_indices):
    vfT = voxel_feats.T
    scT = spatial_coords.T
    bT = batch_ids[None, :]
    combined = _build_combined(vfT, scT, bT)
    g = _sc_gather(combined, peak_indices)
    confT = centroid_confidences.T
    return _tc_affinity(vfT, scT, bT, g, confT, block_n=4096)
